# Initial kernel scaffold; baseline (speedup 1.0000x reference)
import jax
import jax.numpy as jnp
from jax.experimental import pallas as pl


def kernel(x_fix, x_varlen, W_fix, W_var):
    raise NotImplementedError("write your pallas kernel here")


# speedup vs baseline: 34.8980x; 34.8980x over previous
import functools

import jax
import jax.numpy as jnp
from jax import lax
from jax.experimental import pallas as pl
from jax.experimental.pallas import tpu as pltpu
from jax.experimental.pallas import tpu_sc as plsc


def _build_sc_kernel(B, N_FIX, N_VAR, L, D, VOCAB):
    info = plsc.get_sparse_core_info()
    NC, NS = info.num_cores, info.num_subcores
    NW = NC * NS
    per_w = B // NW
    CB = 16
    n_chunks = per_w // CB
    NT = N_FIX + N_VAR
    inv_l = float(1.0 / L)

    mesh = plsc.VectorSubcoreMesh(core_axis_name="c", subcore_axis_name="s")

    @functools.partial(
        pl.kernel,
        mesh=mesh,
        compiler_params=pltpu.CompilerParams(use_tc_tiling_on_sc=False),
        out_type=jax.ShapeDtypeStruct((B * NT, D), jnp.float32),
        scratch_types=[
            pltpu.VMEM((CB * NT,), jnp.int32),
            pltpu.VMEM((CB * L,), jnp.int32),
            pltpu.VMEM((CB * NT, D), jnp.float32),
            pltpu.VMEM((CB * L, D), jnp.float32),
        ],
    )
    def sc_kernel(wfix_hbm, wvar_hbm, mixidx_hbm, varidx_hbm, out_hbm,
                  mixidx_v, varidx_v, outbuf_v, varrows_v):
        wid = lax.axis_index("s") * NC + lax.axis_index("c")

        def chunk_body(c, carry):
            b0 = wid * per_w + c * CB

            pltpu.sync_copy(mixidx_hbm.at[pl.ds(b0 * NT, CB * NT)], mixidx_v)
            pltpu.sync_copy(wfix_hbm.at[mixidx_v], outbuf_v)

            for v in range(N_VAR):
                pltpu.sync_copy(
                    varidx_hbm.at[pl.ds((v * B + b0) * L, CB * L)],
                    varidx_v)
                pltpu.sync_copy(wvar_hbm.at[varidx_v], varrows_v)

                def pool_body(b, carry, v=v):
                    accs = []
                    for h in range(0, D, 16):
                        acc = varrows_v[b * L, pl.ds(h, 16)]
                        for l in range(1, L):
                            acc = acc + varrows_v[b * L + l, pl.ds(h, 16)]
                        accs.append(acc * inv_l)
                    r = b * NT + N_FIX + v
                    for k, h in enumerate(range(0, D, 16)):
                        outbuf_v[r, pl.ds(h, 16)] = accs[k]
                    return carry

                lax.fori_loop(0, CB, pool_body, 0)

            pltpu.sync_copy(outbuf_v, out_hbm.at[pl.ds(b0 * NT, CB * NT)])
            return carry

        lax.fori_loop(0, n_chunks, chunk_body, 0)

    return sc_kernel


def kernel(x_fix, x_varlen, W_fix, W_var):
    B, N_FIX = x_fix.shape
    _, N_VAR, L = x_varlen.shape
    VOCAB, D = W_fix.shape[1], W_fix.shape[2]
    NT = N_FIX + N_VAR

    fix_idx = x_fix.astype(jnp.int32) + (
        jnp.arange(N_FIX, dtype=jnp.int32) * VOCAB)[None, :]
    mix_idx = jnp.concatenate(
        [fix_idx, jnp.zeros((B, N_VAR), jnp.int32)], axis=1)
    var_idx = (x_varlen.astype(jnp.int32) + (
        jnp.arange(N_VAR, dtype=jnp.int32) * VOCAB)[None, :, None]
    ).transpose(1, 0, 2)

    sc_kernel = _build_sc_kernel(B, N_FIX, N_VAR, L, D, VOCAB)
    out = sc_kernel(
        W_fix.reshape(N_FIX * VOCAB, D),
        W_var.reshape(N_VAR * VOCAB, D),
        mix_idx.reshape(-1),
        var_idx.reshape(-1),
    )
    return out.reshape(B, NT * D)
